# baseline (speedup 1.0000x reference)
import functools

import jax
import jax.numpy as jnp
from jax import lax
from jax.experimental import pallas as pl
from jax.experimental.pallas import tpu as pltpu, tpu_sc as plsc

MAX_GOALS = 16384
DEPTH = 16
HALF = DEPTH // 2
D = 256
G = 4096

NC = 2
NS = 16
L = 16
NW = NC * NS
QPW = G // NW
C = 16
NCHUNK = QPW // C
NPAIR = NCHUNK // 2
DV = D // L
UCAP = QPW + L
DUMQ = QPW

_mesh = plsc.VectorSubcoreMesh(
    core_axis_name="c", subcore_axis_name="s", num_cores=NC, num_subcores=NS
)


@functools.partial(
    pl.kernel,
    out_type=(
        jax.ShapeDtypeStruct((G, D), jnp.float32),
        jax.ShapeDtypeStruct((G,), jnp.float32),
    ),
    mesh=_mesh,
    compiler_params=pltpu.CompilerParams(needs_layout_passes=False),
    scratch_types=[
        pltpu.VMEM((QPW,), jnp.int32),
        pltpu.VMEM((QPW,), jnp.int32),
        pltpu.VMEM((QPW,), jnp.int32),
        pltpu.VMEM((UCAP,), jnp.int32),
        pltpu.VMEM((UCAP,), jnp.int32),
        pltpu.VMEM((UCAP,), jnp.int32),
        pltpu.VMEM((UCAP,), jnp.float32),
        pltpu.VMEM((C, HALF, D), jnp.float32),
        pltpu.VMEM((C, HALF, D), jnp.float32),
        pltpu.VMEM((QPW + L, D), jnp.float32),
        pltpu.VMEM((QPW,), jnp.float32),
        pltpu.SemaphoreType.DMA,
        pltpu.SemaphoreType.DMA,
    ],
)
def _tracker(gidx_hbm, cnt_hbm, halves_hbm, sum_hbm, cn_hbm,
             gidx_v, lidx_v, cnt_v, uidx_v, uqid_v, uhi_v, uinv_v,
             buf0_v, buf1_v, acc_v, cn_v, sem0, sem1):
    wid = lax.axis_index("s") * NC + lax.axis_index("c")
    base = wid * QPW

    pltpu.sync_copy(gidx_hbm.at[pl.ds(base, QPW)], gidx_v)
    for t in range(QPW // L):
        g = jnp.clip(gidx_v[pl.ds(t * L, L)], 0, MAX_GOALS - 1)
        gidx_v[pl.ds(t * L, L)] = g
        lidx_v[pl.ds(t * L, L)] = g * 2

    pltpu.async_copy(cnt_hbm.at[gidx_v], cnt_v, sem0).wait()

    zero16 = jnp.zeros((L,), jnp.int32)
    dum16 = jnp.full((L,), DUMQ, jnp.int32)
    for t in range(UCAP // L):
        uidx_v[pl.ds(t * L, L)] = zero16
        uqid_v[pl.ds(t * L, L)] = dum16
        uhi_v[pl.ds(t * L, L)] = zero16
        uinv_v[pl.ds(t * L, L)] = jnp.zeros((L,), jnp.float32)

    iota16 = lax.iota(jnp.int32, L)
    urun = jnp.int32(0)
    for t in range(QPW // L):
        c16 = jnp.minimum(cnt_v[pl.ds(t * L, L)], DEPTH)
        cf = c16.astype(jnp.float32)
        cn_v[pl.ds(t * L, L)] = cf * (1.0 / DEPTH)
        inv16 = 1.0 / jnp.maximum(cf, 1.0)
        hi16 = jnp.maximum(c16 - HALF, 0)
        m = hi16 > 0
        ones = m.astype(jnp.int32)
        incl = plsc.cumsum(ones)
        upos = incl - ones + urun
        g16 = gidx_v[pl.ds(t * L, L)]
        plsc.store_scatter(uidx_v, [upos], g16 * 2 + 1, mask=m)
        plsc.store_scatter(uqid_v, [upos], t * L + iota16, mask=m)
        plsc.store_scatter(uhi_v, [upos], hi16, mask=m)
        plsc.store_scatter(uinv_v, [upos], inv16, mask=m)
        urun = urun + incl[L - 1]
    pltpu.sync_copy(cn_v, cn_hbm.at[pl.ds(base, QPW)])

    bufs = (buf0_v, buf1_v)
    sems = (sem0, sem1)

    def lo_issue(ci, b):
        pltpu.async_copy(
            halves_hbm.at[lidx_v.at[pl.ds(ci * C, C)]], bufs[b], sems[b]
        )

    def lo_drain(ci, b):
        pltpu.make_async_copy(
            halves_hbm.at[lidx_v.at[pl.ds(ci * C, C)]], bufs[b], sems[b]
        ).wait()

    for b in range(2):
        lo_issue(b, b)

    def pair_body(cp, carry):
        for b in range(2):
            ci = cp * 2 + b
            n16 = jnp.minimum(cnt_v[pl.ds(ci * L, L)], DEPTH)
            inv16 = 1.0 / jnp.maximum(n16.astype(jnp.float32), 1.0)
            lo_drain(ci, b)
            buf_v = bufs[b]

            for q in range(C):
                n_s = jnp.minimum(n16[q], HALF)
                inv_b = jnp.full((L,), inv16[q])

                def lo_body(j, acc, q=q, buf_v=buf_v):
                    return tuple(
                        acc[v] + buf_v[q, j, pl.ds(v * L, L)]
                        for v in range(DV)
                    )

                acc0 = tuple(jnp.zeros((L,), jnp.float32) for _ in range(DV))
                acc = lax.fori_loop(0, n_s, lo_body, acc0)
                for v in range(DV):
                    acc_v[ci * C + q, pl.ds(v * L, L)] = acc[v] * inv_b

            @pl.when(cp < NPAIR - 1)
            def _(b=b, cp=cp):
                lo_issue(cp * 2 + 2 + b, b)

        return carry

    lax.fori_loop(0, NPAIR, pair_body, 0)

    nup = urun

    def up_issue(ci, b):
        pltpu.async_copy(
            halves_hbm.at[uidx_v.at[pl.ds(ci * C, C)]], bufs[b], sems[b]
        )

    def up_drain(ci, b):
        pltpu.make_async_copy(
            halves_hbm.at[uidx_v.at[pl.ds(ci * C, C)]], bufs[b], sems[b]
        ).wait()

    for b in range(2):
        @pl.when(b * C < nup)
        def _(b=b):
            up_issue(b, b)

    def up_pair_body(up, carry):
        for b in range(2):
            ci = up * 2 + b
            uq16 = uqid_v[pl.ds(ci * L, L)]
            uh16 = uhi_v[pl.ds(ci * L, L)]
            ui16 = uinv_v[pl.ds(ci * L, L)]

            @pl.when(ci * C < nup)
            def _(b=b, ci=ci):
                up_drain(ci, b)

            buf_v = bufs[b]
            for k in range(C):
                q_s = uq16[k]
                h_s = uh16[k]
                inv_b = jnp.full((L,), ui16[k])

                def hi_body(j, acc, k=k, buf_v=buf_v):
                    return tuple(
                        acc[v] + buf_v[k, j, pl.ds(v * L, L)]
                        for v in range(DV)
                    )

                acc0 = tuple(jnp.zeros((L,), jnp.float32) for _ in range(DV))
                acc = lax.fori_loop(0, h_s, hi_body, acc0)
                for v in range(DV):
                    acc_v[q_s, pl.ds(v * L, L)] = (
                        acc_v[q_s, pl.ds(v * L, L)] + acc[v] * inv_b
                    )

            @pl.when((ci + 2) * C < nup)
            def _(b=b, ci=ci):
                up_issue(ci + 2, b)

        return carry

    nupair = (nup + (2 * C - 1)) >> 5
    lax.fori_loop(0, nupair, up_pair_body, 0)

    pltpu.sync_copy(acc_v.at[pl.ds(0, QPW)], sum_hbm.at[pl.ds(base, QPW)])


def kernel(goal_indices, failed_angles, failed_count):
    halves = failed_angles.reshape(MAX_GOALS * 2, HALF, D)
    summary, count_norm = _tracker(goal_indices, failed_count, halves)
    return summary, count_norm


# speedup vs baseline: 1.4835x; 1.4835x over previous
import functools

import jax
import jax.numpy as jnp
from jax import lax
from jax.experimental import pallas as pl
from jax.experimental.pallas import tpu as pltpu, tpu_sc as plsc

MAX_GOALS = 16384
DEPTH = 16
D = 256
G = 4096

NC = 2
NS = 16
L = 16
NW = NC * NS
QPW = G // NW
C = 8
NCHUNK = QPW // C
NPAIR = NCHUNK // 2
DV = D // L

_mesh = plsc.VectorSubcoreMesh(
    core_axis_name="c", subcore_axis_name="s", num_cores=NC, num_subcores=NS
)


@functools.partial(
    pl.kernel,
    out_type=(
        jax.ShapeDtypeStruct((G, D), jnp.float32),
        jax.ShapeDtypeStruct((G,), jnp.float32),
    ),
    mesh=_mesh,
    scratch_types=[
        pltpu.VMEM((QPW,), jnp.int32),
        pltpu.VMEM((QPW,), jnp.int32),
        pltpu.VMEM((C, DEPTH, D), jnp.float32),
        pltpu.VMEM((C, DEPTH, D), jnp.float32),
        pltpu.VMEM((C, D), jnp.float32),
        pltpu.VMEM((QPW,), jnp.float32),
        pltpu.SemaphoreType.DMA,
        pltpu.SemaphoreType.DMA,
    ],
)
def _tracker(gidx_hbm, cnt_hbm, angles_hbm, sum_hbm, cn_hbm,
             gidx_v, cnt_v, blk0_v, blk1_v, out_v, cn_v, sem0, sem1):
    wid = lax.axis_index("s") * NC + lax.axis_index("c")
    base = wid * QPW

    pltpu.sync_copy(gidx_hbm.at[pl.ds(base, QPW)], gidx_v)
    for t in range(QPW // L):
        g = gidx_v[pl.ds(t * L, L)]
        gidx_v[pl.ds(t * L, L)] = jnp.clip(g, 0, MAX_GOALS - 1)

    pltpu.async_copy(cnt_hbm.at[gidx_v], cnt_v, sem0).wait()

    for t in range(QPW // L):
        nv = jnp.minimum(cnt_v[pl.ds(t * L, L)], DEPTH).astype(jnp.float32)
        cn_v[pl.ds(t * L, L)] = nv * (1.0 / DEPTH)
    pltpu.sync_copy(cn_v, cn_hbm.at[pl.ds(base, QPW)])

    blks = (blk0_v, blk1_v)
    sems = (sem0, sem1)

    def start(ci, b):
        pltpu.async_copy(
            angles_hbm.at[gidx_v.at[pl.ds(ci * C, C)]], blks[b], sems[b]
        )

    start(0, 0)
    start(1, 1)

    def pair_body(ci2, carry):
        n16 = jnp.minimum(cnt_v[pl.ds(ci2 * 2 * C, L)], DEPTH)
        inv16 = 1.0 / jnp.maximum(n16.astype(jnp.float32), 1.0)

        for b in range(2):
            ci = ci2 * 2 + b
            blk_v = blks[b]

            pltpu.make_async_copy(
                angles_hbm.at[gidx_v.at[pl.ds(ci * C, C)]], blk_v, sems[b]
            ).wait()

            for q in range(C):
                n_s = n16[b * C + q]
                inv_b = jnp.full((L,), inv16[b * C + q])

                def row_body(j, acc, q=q, blk_v=blk_v):
                    return tuple(
                        acc[v] + blk_v[q, j, pl.ds(v * L, L)] for v in range(DV)
                    )

                acc0 = tuple(jnp.zeros((L,), jnp.float32) for _ in range(DV))
                acc = lax.fori_loop(0, n_s, row_body, acc0)
                for v in range(DV):
                    out_v[q, pl.ds(v * L, L)] = acc[v] * inv_b

            pltpu.sync_copy(out_v, sum_hbm.at[pl.ds(base + ci * C, C)])

            @pl.when(ci + 2 < NCHUNK)
            def _(ci=ci, b=b):
                start(ci + 2, b)

        return carry

    lax.fori_loop(0, NPAIR, pair_body, 0)


def kernel(goal_indices, failed_angles, failed_count):
    summary, count_norm = _tracker(goal_indices, failed_count, failed_angles)
    return summary, count_norm
